# Initial kernel scaffold; baseline (speedup 1.0000x reference)
import jax
import jax.numpy as jnp
from jax.experimental import pallas as pl

_N = 10000
_NHID = 32
_HEADS = 4
_NGRAPHS = 64


def _encode_body(inp_ref, w_ref, b_ref, o_ref):
    o_ref[...] = jnp.dot(inp_ref[...], w_ref[...].T,
                         preferred_element_type=jnp.float32) + b_ref[...]


def _gat_conv(Xin, src, dst, W_gat, att_src, att_dst, b_gat):
    n = Xin.shape[0]
    loops = jnp.arange(n, dtype=src.dtype)
    s = jnp.concatenate([src, loops])
    d = jnp.concatenate([dst, loops])
    Xp = (Xin @ W_gat.T).reshape(n, _HEADS, _NHID)
    a_s = (Xp * att_src[None, :, :]).sum(-1)
    a_d = (Xp * att_dst[None, :, :]).sum(-1)
    e = a_s[s] + a_d[d]
    e = jnp.where(e > 0, e, 0.2 * e)
    ex = jnp.exp(e)
    den = jax.ops.segment_sum(ex, d, num_segments=n)
    num = jax.ops.segment_sum(Xp[s] * ex[:, :, None], d, num_segments=n)
    out = num / (den[:, :, None] + 1e-16)
    return out.reshape(n, _HEADS * _NHID) + b_gat


def kernel(x, pos, edge_index, batch, W_enc, b_enc, W_res, b_res,
           W_gat, att_src, att_dst, b_gat, W_dec, b_dec):
    inp = jnp.concatenate([x, pos], axis=-1)
    n = inp.shape[0]
    X = pl.pallas_call(
        _encode_body,
        out_shape=jax.ShapeDtypeStruct((n, _NHID), jnp.float32),
    )(inp, W_enc, b_enc)
    src = edge_index[0]
    dst = edge_index[1]
    for _ in range(2):
        conv = _gat_conv(X, src, dst, W_gat, att_src, att_dst, b_gat)
        res = X @ W_res.T + b_res
        X = jax.nn.elu(conv + res).reshape(n, -1, _HEADS).mean(axis=-1)
    Xd = X @ W_dec.T + b_dec
    pooled = jax.ops.segment_sum(Xd, batch, num_segments=_NGRAPHS)
    return pooled.squeeze(-1)


# speedup vs baseline: 100.9686x; 100.9686x over previous
import functools

import jax
import jax.numpy as jnp
from jax import lax
from jax.experimental import pallas as pl
from jax.experimental.pallas import tpu as pltpu
from jax.experimental.pallas import tpu_sc as plsc

_N = 10000
_E = 320000
_NHID = 32
_HEADS = 4
_NGRAPHS = 64
_GW = 48
_ADW = 16
_ZW = 136
_K = 128
_NCHUNK = _E // _K
_ROWS_PER_TILE = _N // 16
_F32 = jnp.float32


def _encode_body(inp_ref, wt_ref, b_ref, vst_ref, vdt_ref,
                 x_ref, g_ref, ad_ref, asad_ref):
    X = jnp.dot(inp_ref[...], wt_ref[...], preferred_element_type=_F32)
    X = X + b_ref[...]
    a_s = jnp.dot(X, vst_ref[...], preferred_element_type=_F32)
    a_d = jnp.dot(X, vdt_ref[...], preferred_element_type=_F32)
    nb = X.shape[0]
    x_ref[...] = X
    g_ref[:, 0:32] = X
    g_ref[:, 32:36] = a_s
    g_ref[:, 36:48] = jnp.zeros((nb, 12), _F32)
    ad_ref[:, 0:4] = a_d
    ad_ref[:, 4:16] = jnp.zeros((nb, 12), _F32)
    asad_ref[:, 0:4] = a_s
    asad_ref[:, 4:8] = a_d


def _combine_body(z0_ref, z1_ref, x_ref, asad_ref, wbd_ref, bgat_ref,
                  wres_ref, bres_ref, vst_ref, vdt_ref, sel_ref, mmat_ref,
                  xn_ref, g_ref, ad_ref, asadn_ref):
    Zs = z0_ref[...] + z1_ref[...]
    X = x_ref[...]
    a_s = asad_ref[:, 0:4]
    a_d = asad_ref[:, 4:8]
    es = a_s + a_d
    es = jnp.where(es > 0.0, es, 0.2 * es)
    wself = jnp.exp(es)
    sel = sel_ref[...]
    w128 = jnp.dot(wself, sel, preferred_element_type=_F32)
    xt = jnp.concatenate([X, X, X, X], axis=1)
    zfull = Zs[:, 0:128] + w128 * xt
    den = Zs[:, 128:132] + wself
    num = jnp.dot(zfull, wbd_ref[...], preferred_element_type=_F32)
    den128 = jnp.dot(den, sel, preferred_element_type=_F32)
    conv = num / (den128 + 1e-16) + bgat_ref[...]
    res = jnp.dot(X, wres_ref[...], preferred_element_type=_F32) + bres_ref[...]
    t = conv + res
    t = jnp.where(t > 0.0, t, jnp.exp(t) - 1.0)
    Xn = jnp.dot(t, mmat_ref[...], preferred_element_type=_F32)
    a_sn = jnp.dot(Xn, vst_ref[...], preferred_element_type=_F32)
    a_dn = jnp.dot(Xn, vdt_ref[...], preferred_element_type=_F32)
    nb = Xn.shape[0]
    xn_ref[...] = Xn
    g_ref[:, 0:32] = Xn
    g_ref[:, 32:36] = a_sn
    g_ref[:, 36:48] = jnp.zeros((nb, 12), _F32)
    ad_ref[:, 0:4] = a_dn
    ad_ref[:, 4:16] = jnp.zeros((nb, 12), _F32)
    asadn_ref[:, 0:4] = a_sn
    asadn_ref[:, 4:8] = a_dn


def _decode_body(x_ref, batch_ref, wd_ref, bd_ref, o_ref):
    s = jnp.dot(x_ref[...], wd_ref[...], preferred_element_type=_F32)
    gid = lax.broadcasted_iota(jnp.int32, (_N, _NGRAPHS), 1)
    oh = (batch_ref[...] == gid).astype(_F32)
    pooled = lax.dot_general(s, oh, (((0,), (0,)), ((), ())),
                             preferred_element_type=_F32)
    counts = jnp.sum(oh, axis=0, keepdims=True)
    o_ref[...] = pooled + bd_ref[0, 0] * counts


@functools.partial(
    pl.kernel,
    out_type=jax.ShapeDtypeStruct((2, _N, _ZW), _F32),
    mesh=plsc.VectorSubcoreMesh(core_axis_name="c", subcore_axis_name="s"),
    scratch_types=[
        pltpu.VMEM((2, _K), jnp.int32),
        pltpu.VMEM((_K, _GW), _F32),
        pltpu.VMEM((_K, _ADW), _F32),
        pltpu.VMEM((_K, _ZW), _F32),
        pltpu.VMEM_SHARED((_N, _ZW), _F32),
        pltpu.SemaphoreType.DMA,
        pltpu.SemaphoreType.DMA,
    ],
    compiler_params=pltpu.CompilerParams(use_tc_tiling_on_sc=False,
                                         needs_layout_passes=False),
)
def _sc_edge_kernel(g_hbm, ad_hbm, ei_hbm, out_hbm,
                    ei_v, g_rows, ad_rows, wz, acc, sem1, sem2):
    c = lax.axis_index("c")
    s = lax.axis_index("s")
    tile = c * 16 + s

    @pl.loop(0, _K)
    def _zero_wz(e):
        for j in (0, 16, 32, 48, 64, 80, 96, 112, 120):
            wz[e, pl.ds(j, 16)] = jnp.zeros((16,), _F32)

    base_row = s * _ROWS_PER_TILE
    nfull = _ROWS_PER_TILE // _K
    rem = _ROWS_PER_TILE % _K

    @pl.loop(0, nfull)
    def _zero_acc(i):
        pltpu.sync_copy(wz, acc.at[pl.ds(base_row + i * _K, _K)])

    pltpu.sync_copy(wz.at[pl.ds(0, rem)],
                    acc.at[pl.ds(base_row + nfull * _K, rem)])
    plsc.subcore_barrier()

    lanes = lax.iota(jnp.int32, 16)
    e4 = lanes // 4
    h4 = lanes - 4 * e4

    ntiles_extra = _NCHUNK - 78 * 32
    cnt = jnp.where(tile < ntiles_extra, 79, 78)
    start = 78 * tile + jnp.minimum(tile, ntiles_extra)

    @pl.loop(0, cnt)
    def _chunk(i):
        ebase = (start + i) * _K
        pltpu.sync_copy(ei_hbm.at[:, pl.ds(ebase, _K)], ei_v)
        cp1 = pltpu.async_copy(g_hbm.at[ei_v.at[0]], g_rows, sem1)
        cp2 = pltpu.async_copy(ad_hbm.at[ei_v.at[1]], ad_rows, sem2)
        cp1.wait()
        cp2.wait()

        @pl.loop(0, _K // 4)
        def _wgrp(g):
            row_idx = g * 4 + e4
            a_s = plsc.load_gather(g_rows, [row_idx, 32 + h4])
            a_d = plsc.load_gather(ad_rows, [row_idx, h4])
            e = a_s + a_d
            e = jnp.where(e > 0.0, e, 0.2 * e)
            plsc.store_scatter(wz, [row_idx, 128 + h4], jnp.exp(e))

        @pl.loop(0, _K)
        def _wrow(e):
            x0 = g_rows[e, pl.ds(0, 16)]
            x1 = g_rows[e, pl.ds(16, 16)]
            wv = wz[e, pl.ds(120, 16)]
            for h in range(4):
                wh = wv[8 + h]
                wz[e, pl.ds(h * 32, 16)] = x0 * wh
                wz[e, pl.ds(h * 32 + 16, 16)] = x1 * wh

        pltpu.sync_copy(wz, acc.at[ei_v.at[1]], add=True)

    plsc.subcore_barrier()
    pltpu.sync_copy(acc.at[pl.ds(base_row, _ROWS_PER_TILE)],
                    out_hbm.at[c, pl.ds(base_row, _ROWS_PER_TILE)])


def kernel(x, pos, edge_index, batch, W_enc, b_enc, W_res, b_res,
           W_gat, att_src, att_dst, b_gat, W_dec, b_dec):
    Wg3 = W_gat.reshape(_HEADS, _NHID, _NHID)
    vst = jnp.einsum("hj,hjk->kh", att_src, Wg3)
    vdt = jnp.einsum("hj,hjk->kh", att_dst, Wg3)
    wbd = jax.scipy.linalg.block_diag(*[Wg3[h].T for h in range(_HEADS)])
    sel = jnp.repeat(jnp.eye(_HEADS, dtype=_F32), _NHID, axis=1)
    mmat = jnp.repeat(jnp.eye(_NHID, dtype=_F32), _HEADS, axis=0) * 0.25
    bgat = b_gat.reshape(1, -1)
    bres = b_res.reshape(1, -1)
    wres_t = W_res.T

    inp = jnp.concatenate([x, pos], axis=1)
    node_shapes = (
        jax.ShapeDtypeStruct((_N, _NHID), _F32),
        jax.ShapeDtypeStruct((_N, _GW), _F32),
        jax.ShapeDtypeStruct((_N, _ADW), _F32),
        jax.ShapeDtypeStruct((_N, 8), _F32),
    )
    _NB = 2000
    _GRID = _N // _NB

    def _rows(w):
        return pl.BlockSpec((_NB, w), lambda i: (i, 0))

    def _full(a, b):
        return pl.BlockSpec((a, b), lambda i: (0, 0))

    node_out_specs = (_rows(_NHID), _rows(_GW), _rows(_ADW), _rows(8))
    X, G, AD, ASAD = pl.pallas_call(
        _encode_body, out_shape=node_shapes,
        grid=(_GRID,),
        in_specs=[_rows(128), _full(128, _NHID), _full(1, _NHID),
                  _full(_NHID, _HEADS), _full(_NHID, _HEADS)],
        out_specs=node_out_specs,
    )(inp, W_enc.T, b_enc.reshape(1, -1), vst, vdt)

    for _ in range(2):
        zpart = _sc_edge_kernel(G, AD, edge_index)
        X, G, AD, ASAD = pl.pallas_call(
            _combine_body, out_shape=node_shapes,
            grid=(_GRID,),
            in_specs=[_rows(_ZW), _rows(_ZW), _rows(_NHID), _rows(8),
                      _full(128, 128), _full(1, 128), _full(_NHID, 128),
                      _full(1, 128), _full(_NHID, _HEADS),
                      _full(_NHID, _HEADS), _full(_HEADS, 128),
                      _full(128, _NHID)],
            out_specs=node_out_specs,
        )(zpart[0], zpart[1], X, ASAD, wbd, bgat, wres_t, bres, vst, vdt, sel, mmat)

    out = pl.pallas_call(
        _decode_body,
        out_shape=jax.ShapeDtypeStruct((1, _NGRAPHS), _F32),
    )(X, batch.reshape(-1, 1), W_dec.T, b_dec.reshape(1, 1))
    return out.reshape(_NGRAPHS)


# speedup vs baseline: 124.5120x; 1.2332x over previous
import functools

import jax
import jax.numpy as jnp
from jax import lax
from jax.experimental import pallas as pl
from jax.experimental.pallas import tpu as pltpu
from jax.experimental.pallas import tpu_sc as plsc

_N = 10000
_E = 320000
_NHID = 32
_HEADS = 4
_NGRAPHS = 64
_GW = 48
_ADW = 16
_ZW = 136
_K = 128
_NCHUNK = _E // _K
_ROWS_PER_TILE = _N // 16
_F32 = jnp.float32


def _encode_body(inp_ref, wt_ref, b_ref, vst_ref, vdt_ref,
                 x_ref, g_ref, ad_ref, asad_ref):
    X = jnp.dot(inp_ref[...], wt_ref[...], preferred_element_type=_F32)
    X = X + b_ref[...]
    a_s = jnp.dot(X, vst_ref[...], preferred_element_type=_F32)
    a_d = jnp.dot(X, vdt_ref[...], preferred_element_type=_F32)
    nb = X.shape[0]
    x_ref[...] = X
    g_ref[:, 0:32] = X
    g_ref[:, 32:36] = a_s
    g_ref[:, 36:48] = jnp.zeros((nb, 12), _F32)
    ad_ref[:, 0:4] = a_d
    ad_ref[:, 4:16] = jnp.zeros((nb, 12), _F32)
    asad_ref[:, 0:4] = a_s
    asad_ref[:, 4:8] = a_d


def _combine_body(z0_ref, z1_ref, x_ref, asad_ref, wbd_ref, bgat_ref,
                  wres_ref, bres_ref, vst_ref, vdt_ref, sel_ref, mmat_ref,
                  xn_ref, g_ref, ad_ref, asadn_ref):
    Zs = z0_ref[...] + z1_ref[...]
    X = x_ref[...]
    a_s = asad_ref[:, 0:4]
    a_d = asad_ref[:, 4:8]
    es = a_s + a_d
    es = jnp.where(es > 0.0, es, 0.2 * es)
    wself = jnp.exp(es)
    sel = sel_ref[...]
    w128 = jnp.dot(wself, sel, preferred_element_type=_F32)
    xt = jnp.concatenate([X, X, X, X], axis=1)
    zfull = Zs[:, 0:128] + w128 * xt
    den = Zs[:, 128:132] + wself
    num = jnp.dot(zfull, wbd_ref[...], preferred_element_type=_F32)
    den128 = jnp.dot(den, sel, preferred_element_type=_F32)
    conv = num / (den128 + 1e-16) + bgat_ref[...]
    res = jnp.dot(X, wres_ref[...], preferred_element_type=_F32) + bres_ref[...]
    t = conv + res
    t = jnp.where(t > 0.0, t, jnp.exp(t) - 1.0)
    Xn = jnp.dot(t, mmat_ref[...], preferred_element_type=_F32)
    a_sn = jnp.dot(Xn, vst_ref[...], preferred_element_type=_F32)
    a_dn = jnp.dot(Xn, vdt_ref[...], preferred_element_type=_F32)
    nb = Xn.shape[0]
    xn_ref[...] = Xn
    g_ref[:, 0:32] = Xn
    g_ref[:, 32:36] = a_sn
    g_ref[:, 36:48] = jnp.zeros((nb, 12), _F32)
    ad_ref[:, 0:4] = a_dn
    ad_ref[:, 4:16] = jnp.zeros((nb, 12), _F32)
    asadn_ref[:, 0:4] = a_sn
    asadn_ref[:, 4:8] = a_dn


def _decode_body(x_ref, batch_ref, wd_ref, bd_ref, o_ref):
    s = jnp.dot(x_ref[...], wd_ref[...], preferred_element_type=_F32)
    gid = lax.broadcasted_iota(jnp.int32, (_N, _NGRAPHS), 1)
    oh = (batch_ref[...] == gid).astype(_F32)
    pooled = lax.dot_general(s, oh, (((0,), (0,)), ((), ())),
                             preferred_element_type=_F32)
    counts = jnp.sum(oh, axis=0, keepdims=True)
    o_ref[...] = pooled + bd_ref[0, 0] * counts


@functools.partial(
    pl.kernel,
    out_type=jax.ShapeDtypeStruct((2, _N, _ZW), _F32),
    mesh=plsc.VectorSubcoreMesh(core_axis_name="c", subcore_axis_name="s"),
    scratch_types=[
        pltpu.VMEM((2, _K), jnp.int32),
        pltpu.VMEM((2, _K), jnp.int32),
        pltpu.VMEM((_K, _GW), _F32),
        pltpu.VMEM((_K, _GW), _F32),
        pltpu.VMEM((_K, _ADW), _F32),
        pltpu.VMEM((_K, _ADW), _F32),
        pltpu.VMEM((_K, _ZW), _F32),
        pltpu.VMEM_SHARED((_N, _ZW), _F32),
        pltpu.SemaphoreType.DMA,
        pltpu.SemaphoreType.DMA,
        pltpu.SemaphoreType.DMA,
        pltpu.SemaphoreType.DMA,
        pltpu.SemaphoreType.DMA,
        pltpu.SemaphoreType.DMA,
    ],
    compiler_params=pltpu.CompilerParams(use_tc_tiling_on_sc=False,
                                         needs_layout_passes=False),
)
def _sc_edge_kernel(g_hbm, ad_hbm, ei_hbm, out_hbm,
                    ei0, ei1, g0, g1, ad0, ad1, wz, acc,
                    sei0, sei1, sg0, sg1, sad0, sad1):
    c = lax.axis_index("c")
    s = lax.axis_index("s")
    tile = c * 16 + s
    eib = (ei0, ei1)
    gb = (g0, g1)
    adb = (ad0, ad1)
    seib = (sei0, sei1)
    sgb = (sg0, sg1)
    sadb = (sad0, sad1)

    @pl.loop(0, _K)
    def _zero_wz(e):
        for j in (0, 16, 32, 48, 64, 80, 96, 112, 120):
            wz[e, pl.ds(j, 16)] = jnp.zeros((16,), _F32)

    base_row = s * _ROWS_PER_TILE
    nfull = _ROWS_PER_TILE // _K
    rem = _ROWS_PER_TILE % _K

    @pl.loop(0, nfull)
    def _zero_acc(i):
        pltpu.sync_copy(wz, acc.at[pl.ds(base_row + i * _K, _K)])

    pltpu.sync_copy(wz.at[pl.ds(0, rem)],
                    acc.at[pl.ds(base_row + nfull * _K, rem)])
    plsc.subcore_barrier()

    lanes = lax.iota(jnp.int32, 16)
    e4 = lanes // 4
    h4 = lanes - 4 * e4

    cnt = jnp.where(tile < 2, 80, 78)
    start = 78 * tile + 2 * jnp.minimum(tile, 2)

    def _load_ei(ch, b):
        pltpu.async_copy(ei_hbm.at[:, pl.ds(ch * _K, _K)], eib[b], seib[b])

    def _wait_ei(b):
        pltpu.make_async_copy(ei_hbm.at[:, pl.ds(0, _K)], eib[b],
                              seib[b]).wait()

    def _start_gathers(b):
        pltpu.async_copy(g_hbm.at[eib[b].at[0]], gb[b], sgb[b])
        pltpu.async_copy(ad_hbm.at[eib[b].at[1]], adb[b], sadb[b])

    def _wait_gathers(b):
        pltpu.make_async_copy(g_hbm.at[eib[b].at[0]], gb[b], sgb[b]).wait()
        pltpu.make_async_copy(ad_hbm.at[eib[b].at[1]], adb[b], sadb[b]).wait()

    _load_ei(start, 0)
    _wait_ei(0)
    _start_gathers(0)
    _load_ei(start + 1, 1)

    @pl.loop(0, cnt, step=2)
    def _chunk(i):
        for b in (0, 1):
            cur = i + b
            _wait_gathers(b)

            @pl.when(cur + 1 < cnt)
            def _prefetch():
                _wait_ei(1 - b)
                _start_gathers(1 - b)

            @pl.loop(0, _K // 4, unroll=2)
            def _wgrp(g):
                row_idx = g * 4 + e4
                a_s = plsc.load_gather(gb[b], [row_idx, 32 + h4])
                a_d = plsc.load_gather(adb[b], [row_idx, h4])
                e = a_s + a_d
                e = jnp.where(e > 0.0, e, 0.2 * e)
                plsc.store_scatter(wz, [row_idx, 128 + h4], jnp.exp(e))

            @pl.loop(0, _K, unroll=2)
            def _wrow(e):
                x0 = gb[b][e, pl.ds(0, 16)]
                x1 = gb[b][e, pl.ds(16, 16)]
                wv = wz[e, pl.ds(120, 16)]
                for h in range(4):
                    wh = wv[8 + h]
                    wz[e, pl.ds(h * 32, 16)] = x0 * wh
                    wz[e, pl.ds(h * 32 + 16, 16)] = x1 * wh

            pltpu.sync_copy(wz, acc.at[eib[b].at[1]], add=True)

            @pl.when(cur + 2 < cnt)
            def _next_ei():
                _load_ei(start + cur + 2, b)

    plsc.subcore_barrier()
    pltpu.sync_copy(acc.at[pl.ds(base_row, _ROWS_PER_TILE)],
                    out_hbm.at[c, pl.ds(base_row, _ROWS_PER_TILE)])


def kernel(x, pos, edge_index, batch, W_enc, b_enc, W_res, b_res,
           W_gat, att_src, att_dst, b_gat, W_dec, b_dec):
    Wg3 = W_gat.reshape(_HEADS, _NHID, _NHID)
    vst = jnp.einsum("hj,hjk->kh", att_src, Wg3)
    vdt = jnp.einsum("hj,hjk->kh", att_dst, Wg3)
    wbd = jax.scipy.linalg.block_diag(*[Wg3[h].T for h in range(_HEADS)])
    sel = jnp.repeat(jnp.eye(_HEADS, dtype=_F32), _NHID, axis=1)
    mmat = jnp.repeat(jnp.eye(_NHID, dtype=_F32), _HEADS, axis=0) * 0.25
    bgat = b_gat.reshape(1, -1)
    bres = b_res.reshape(1, -1)
    wres_t = W_res.T

    inp = jnp.concatenate([x, pos], axis=1)
    node_shapes = (
        jax.ShapeDtypeStruct((_N, _NHID), _F32),
        jax.ShapeDtypeStruct((_N, _GW), _F32),
        jax.ShapeDtypeStruct((_N, _ADW), _F32),
        jax.ShapeDtypeStruct((_N, 8), _F32),
    )
    _NB = 2000
    _GRID = _N // _NB

    def _rows(w):
        return pl.BlockSpec((_NB, w), lambda i: (i, 0))

    def _full(a, b):
        return pl.BlockSpec((a, b), lambda i: (0, 0))

    node_out_specs = (_rows(_NHID), _rows(_GW), _rows(_ADW), _rows(8))
    X, G, AD, ASAD = pl.pallas_call(
        _encode_body, out_shape=node_shapes,
        grid=(_GRID,),
        in_specs=[_rows(128), _full(128, _NHID), _full(1, _NHID),
                  _full(_NHID, _HEADS), _full(_NHID, _HEADS)],
        out_specs=node_out_specs,
    )(inp, W_enc.T, b_enc.reshape(1, -1), vst, vdt)

    for _ in range(2):
        zpart = _sc_edge_kernel(G, AD, edge_index)
        X, G, AD, ASAD = pl.pallas_call(
            _combine_body, out_shape=node_shapes,
            grid=(_GRID,),
            in_specs=[_rows(_ZW), _rows(_ZW), _rows(_NHID), _rows(8),
                      _full(128, 128), _full(1, 128), _full(_NHID, 128),
                      _full(1, 128), _full(_NHID, _HEADS),
                      _full(_NHID, _HEADS), _full(_HEADS, 128),
                      _full(128, _NHID)],
            out_specs=node_out_specs,
        )(zpart[0], zpart[1], X, ASAD, wbd, bgat, wres_t, bres, vst, vdt, sel, mmat)

    out = pl.pallas_call(
        _decode_body,
        out_shape=jax.ShapeDtypeStruct((1, _NGRAPHS), _F32),
    )(X, batch.reshape(-1, 1), W_dec.T, b_dec.reshape(1, 1))
    return out.reshape(_NGRAPHS)


# speedup vs baseline: 129.7788x; 1.0423x over previous
import functools

import jax
import jax.numpy as jnp
from jax import lax
from jax.experimental import pallas as pl
from jax.experimental.pallas import tpu as pltpu
from jax.experimental.pallas import tpu_sc as plsc

_N = 10000
_E = 320000
_NHID = 32
_HEADS = 4
_NGRAPHS = 64
_GW = 48
_ADW = 16
_ZW = 136
_K = 128
_NCHUNK = _E // _K
_ROWS_PER_TILE = _N // 16
_F32 = jnp.float32


def _encode_body(inp_ref, wt_ref, b_ref, vst_ref, vdt_ref,
                 x_ref, g_ref, ad_ref, asad_ref):
    X = jnp.dot(inp_ref[...], wt_ref[...], preferred_element_type=_F32)
    X = X + b_ref[...]
    a_s = jnp.dot(X, vst_ref[...], preferred_element_type=_F32)
    a_d = jnp.dot(X, vdt_ref[...], preferred_element_type=_F32)
    nb = X.shape[0]
    x_ref[...] = X
    g_ref[:, 0:32] = X
    g_ref[:, 32:36] = a_s
    g_ref[:, 36:48] = jnp.zeros((nb, 12), _F32)
    ad_ref[:, 0:4] = a_d
    ad_ref[:, 4:16] = jnp.zeros((nb, 12), _F32)
    asad_ref[:, 0:4] = a_s
    asad_ref[:, 4:8] = a_d


def _combine_body(z0_ref, z1_ref, x_ref, asad_ref, wbd_ref, bgat_ref,
                  wres_ref, bres_ref, vst_ref, vdt_ref, sel_ref, mmat_ref,
                  xn_ref, g_ref, ad_ref, asadn_ref):
    Zs = z0_ref[0] + z1_ref[0]
    X = x_ref[...]
    a_s = asad_ref[:, 0:4]
    a_d = asad_ref[:, 4:8]
    es = a_s + a_d
    es = jnp.where(es > 0.0, es, 0.2 * es)
    wself = jnp.exp(es)
    sel = sel_ref[...]
    w128 = jnp.dot(wself, sel, preferred_element_type=_F32)
    xt = jnp.concatenate([X, X, X, X], axis=1)
    zfull = Zs[:, 0:128] + w128 * xt
    den = Zs[:, 128:132] + wself
    num = jnp.dot(zfull, wbd_ref[...], preferred_element_type=_F32)
    den128 = jnp.dot(den, sel, preferred_element_type=_F32)
    conv = num / (den128 + 1e-16) + bgat_ref[...]
    res = jnp.dot(X, wres_ref[...], preferred_element_type=_F32) + bres_ref[...]
    t = conv + res
    t = jnp.where(t > 0.0, t, jnp.exp(t) - 1.0)
    Xn = jnp.dot(t, mmat_ref[...], preferred_element_type=_F32)
    a_sn = jnp.dot(Xn, vst_ref[...], preferred_element_type=_F32)
    a_dn = jnp.dot(Xn, vdt_ref[...], preferred_element_type=_F32)
    nb = Xn.shape[0]
    xn_ref[...] = Xn
    g_ref[:, 0:32] = Xn
    g_ref[:, 32:36] = a_sn
    g_ref[:, 36:48] = jnp.zeros((nb, 12), _F32)
    ad_ref[:, 0:4] = a_dn
    ad_ref[:, 4:16] = jnp.zeros((nb, 12), _F32)
    asadn_ref[:, 0:4] = a_sn
    asadn_ref[:, 4:8] = a_dn


def _decode_body(x_ref, batch_ref, wd_ref, bd_ref, o_ref):
    s = jnp.dot(x_ref[...], wd_ref[...], preferred_element_type=_F32)
    gid = lax.broadcasted_iota(jnp.int32, (_N, _NGRAPHS), 1)
    oh = (batch_ref[...] == gid).astype(_F32)
    pooled = lax.dot_general(s, oh, (((0,), (0,)), ((), ())),
                             preferred_element_type=_F32)
    counts = jnp.sum(oh, axis=0, keepdims=True)
    o_ref[...] = pooled + bd_ref[0, 0] * counts


@functools.partial(
    pl.kernel,
    out_type=jax.ShapeDtypeStruct((2, _N, _ZW), _F32),
    mesh=plsc.VectorSubcoreMesh(core_axis_name="c", subcore_axis_name="s"),
    scratch_types=[
        pltpu.VMEM((2, _K), jnp.int32),
        pltpu.VMEM((2, _K), jnp.int32),
        pltpu.VMEM((_K, _GW), _F32),
        pltpu.VMEM((_K, _GW), _F32),
        pltpu.VMEM((_K, _ADW), _F32),
        pltpu.VMEM((_K, _ADW), _F32),
        pltpu.VMEM((_K, _ZW), _F32),
        pltpu.VMEM_SHARED((_N, _ZW), _F32),
        pltpu.SemaphoreType.DMA,
        pltpu.SemaphoreType.DMA,
        pltpu.SemaphoreType.DMA,
        pltpu.SemaphoreType.DMA,
        pltpu.SemaphoreType.DMA,
        pltpu.SemaphoreType.DMA,
    ],
    compiler_params=pltpu.CompilerParams(use_tc_tiling_on_sc=False,
                                         needs_layout_passes=False),
)
def _sc_edge_kernel(g_hbm, ad_hbm, ei_hbm, out_hbm,
                    ei0, ei1, g0, g1, ad0, ad1, wz, acc,
                    sei0, sei1, sg0, sg1, sad0, sad1):
    c = lax.axis_index("c")
    s = lax.axis_index("s")
    tile = c * 16 + s
    eib = (ei0, ei1)
    gb = (g0, g1)
    adb = (ad0, ad1)
    seib = (sei0, sei1)
    sgb = (sg0, sg1)
    sadb = (sad0, sad1)

    @pl.loop(0, _K)
    def _zero_wz(e):
        for j in (0, 16, 32, 48, 64, 80, 96, 112, 120):
            wz[e, pl.ds(j, 16)] = jnp.zeros((16,), _F32)

    base_row = s * _ROWS_PER_TILE
    nfull = _ROWS_PER_TILE // _K
    rem = _ROWS_PER_TILE % _K

    @pl.loop(0, nfull)
    def _zero_acc(i):
        pltpu.sync_copy(wz, acc.at[pl.ds(base_row + i * _K, _K)])

    pltpu.sync_copy(wz.at[pl.ds(0, rem)],
                    acc.at[pl.ds(base_row + nfull * _K, rem)])
    plsc.subcore_barrier()

    lanes = lax.iota(jnp.int32, 16)
    e4 = lanes // 4
    h4 = lanes - 4 * e4

    cnt = jnp.where(tile < 2, 80, 78)
    start = 78 * tile + 2 * jnp.minimum(tile, 2)

    def _load_ei(ch, b):
        pltpu.async_copy(ei_hbm.at[:, pl.ds(ch * _K, _K)], eib[b], seib[b])

    def _wait_ei(b):
        pltpu.make_async_copy(ei_hbm.at[:, pl.ds(0, _K)], eib[b],
                              seib[b]).wait()

    def _start_gathers(b):
        pltpu.async_copy(g_hbm.at[eib[b].at[0]], gb[b], sgb[b])
        pltpu.async_copy(ad_hbm.at[eib[b].at[1]], adb[b], sadb[b])

    def _wait_gathers(b):
        pltpu.make_async_copy(g_hbm.at[eib[b].at[0]], gb[b], sgb[b]).wait()
        pltpu.make_async_copy(ad_hbm.at[eib[b].at[1]], adb[b], sadb[b]).wait()

    _load_ei(start, 0)
    _wait_ei(0)
    _start_gathers(0)
    _load_ei(start + 1, 1)

    @pl.loop(0, cnt, step=2)
    def _chunk(i):
        for b in (0, 1):
            cur = i + b
            _wait_gathers(b)

            @pl.when(cur + 1 < cnt)
            def _prefetch():
                _wait_ei(1 - b)
                _start_gathers(1 - b)

            @pl.loop(0, _K // 4, unroll=4)
            def _wgrp(g):
                row_idx = g * 4 + e4
                a_s = plsc.load_gather(gb[b], [row_idx, 32 + h4])
                a_d = plsc.load_gather(adb[b], [row_idx, h4])
                e = a_s + a_d
                e = jnp.where(e > 0.0, e, 0.2 * e)
                plsc.store_scatter(wz, [row_idx, 128 + h4], jnp.exp(e))

            @pl.loop(0, _K, unroll=4)
            def _wrow(e):
                x0 = gb[b][e, pl.ds(0, 16)]
                x1 = gb[b][e, pl.ds(16, 16)]
                wv = wz[e, pl.ds(120, 16)]
                for h in range(4):
                    wh = wv[8 + h]
                    wz[e, pl.ds(h * 32, 16)] = x0 * wh
                    wz[e, pl.ds(h * 32 + 16, 16)] = x1 * wh

            pltpu.sync_copy(wz, acc.at[eib[b].at[1]], add=True)

            @pl.when(cur + 2 < cnt)
            def _next_ei():
                _load_ei(start + cur + 2, b)

    plsc.subcore_barrier()
    pltpu.sync_copy(acc.at[pl.ds(base_row, _ROWS_PER_TILE)],
                    out_hbm.at[c, pl.ds(base_row, _ROWS_PER_TILE)])


def kernel(x, pos, edge_index, batch, W_enc, b_enc, W_res, b_res,
           W_gat, att_src, att_dst, b_gat, W_dec, b_dec):
    Wg3 = W_gat.reshape(_HEADS, _NHID, _NHID)
    vst = jnp.einsum("hj,hjk->kh", att_src, Wg3)
    vdt = jnp.einsum("hj,hjk->kh", att_dst, Wg3)
    wbd = jax.scipy.linalg.block_diag(*[Wg3[h].T for h in range(_HEADS)])
    sel = jnp.repeat(jnp.eye(_HEADS, dtype=_F32), _NHID, axis=1)
    mmat = jnp.repeat(jnp.eye(_NHID, dtype=_F32), _HEADS, axis=0) * 0.25
    bgat = b_gat.reshape(1, -1)
    bres = b_res.reshape(1, -1)
    wres_t = W_res.T

    inp = jnp.concatenate([x, pos], axis=1)
    node_shapes = (
        jax.ShapeDtypeStruct((_N, _NHID), _F32),
        jax.ShapeDtypeStruct((_N, _GW), _F32),
        jax.ShapeDtypeStruct((_N, _ADW), _F32),
        jax.ShapeDtypeStruct((_N, 8), _F32),
    )
    _NB = 2000
    _GRID = _N // _NB

    def _rows(w):
        return pl.BlockSpec((_NB, w), lambda i: (i, 0))

    def _full(a, b):
        return pl.BlockSpec((a, b), lambda i: (0, 0))

    node_out_specs = (_rows(_NHID), _rows(_GW), _rows(_ADW), _rows(8))
    X, G, AD, ASAD = pl.pallas_call(
        _encode_body, out_shape=node_shapes,
        grid=(_GRID,),
        in_specs=[_rows(128), _full(128, _NHID), _full(1, _NHID),
                  _full(_NHID, _HEADS), _full(_NHID, _HEADS)],
        out_specs=node_out_specs,
    )(inp, W_enc.T, b_enc.reshape(1, -1), vst, vdt)

    for _ in range(2):
        zpart = _sc_edge_kernel(G, AD, edge_index)
        X, G, AD, ASAD = pl.pallas_call(
            _combine_body, out_shape=node_shapes,
            grid=(_GRID,),
            in_specs=[pl.BlockSpec((1, _NB, _ZW), lambda i: (0, i, 0)),
                      pl.BlockSpec((1, _NB, _ZW), lambda i: (1, i, 0)),
                      _rows(_NHID), _rows(8),
                      _full(128, 128), _full(1, 128), _full(_NHID, 128),
                      _full(1, 128), _full(_NHID, _HEADS),
                      _full(_NHID, _HEADS), _full(_HEADS, 128),
                      _full(128, _NHID)],
            out_specs=node_out_specs,
        )(zpart, zpart, X, ASAD, wbd, bgat, wres_t, bres, vst, vdt, sel, mmat)

    out = pl.pallas_call(
        _decode_body,
        out_shape=jax.ShapeDtypeStruct((1, _NGRAPHS), _F32),
    )(X, batch.reshape(-1, 1), W_dec.T, b_dec.reshape(1, 1))
    return out.reshape(_NGRAPHS)


# speedup vs baseline: 140.6330x; 1.0836x over previous
import functools

import jax
import jax.numpy as jnp
from jax import lax
from jax.experimental import pallas as pl
from jax.experimental.pallas import tpu as pltpu
from jax.experimental.pallas import tpu_sc as plsc

_N = 10000
_E = 320000
_NHID = 32
_HEADS = 4
_NGRAPHS = 64
_GW = 40
_ADW = 16
_ZW = 136
_K = 64
_NCHUNK = _E // _K
_ROWS_PER_TILE = _N // 16
_F32 = jnp.float32


def _encode_body(inp_ref, wt_ref, b_ref, vst_ref, vdt_ref,
                 x_ref, g_ref, ad_ref, asad_ref):
    X = jnp.dot(inp_ref[...], wt_ref[...], preferred_element_type=_F32)
    X = X + b_ref[...]
    a_s = jnp.dot(X, vst_ref[...], preferred_element_type=_F32)
    a_d = jnp.dot(X, vdt_ref[...], preferred_element_type=_F32)
    nb = X.shape[0]
    x_ref[...] = X
    g_ref[:, 0:32] = X
    g_ref[:, 32:36] = a_s
    g_ref[:, 36:40] = jnp.zeros((nb, 4), _F32)
    ad_ref[:, 0:4] = a_d
    ad_ref[:, 4:16] = jnp.zeros((nb, 12), _F32)
    asad_ref[:, 0:4] = a_s
    asad_ref[:, 4:8] = a_d


def _combine_body(z0_ref, z1_ref, x_ref, asad_ref, wbd_ref, bgat_ref,
                  wres_ref, bres_ref, vst_ref, vdt_ref, sel_ref, mmat_ref,
                  xn_ref, g_ref, ad_ref, asadn_ref):
    Zs = z0_ref[0] + z1_ref[0]
    X = x_ref[...]
    a_s = asad_ref[:, 0:4]
    a_d = asad_ref[:, 4:8]
    es = a_s + a_d
    es = jnp.where(es > 0.0, es, 0.2 * es)
    wself = jnp.exp(es)
    sel = sel_ref[...]
    w128 = jnp.dot(wself, sel, preferred_element_type=_F32)
    xt = jnp.concatenate([X, X, X, X], axis=1)
    zfull = Zs[:, 0:128] + w128 * xt
    den = Zs[:, 128:132] + wself
    num = jnp.dot(zfull, wbd_ref[...], preferred_element_type=_F32)
    den128 = jnp.dot(den, sel, preferred_element_type=_F32)
    conv = num / (den128 + 1e-16) + bgat_ref[...]
    res = jnp.dot(X, wres_ref[...], preferred_element_type=_F32) + bres_ref[...]
    t = conv + res
    t = jnp.where(t > 0.0, t, jnp.exp(t) - 1.0)
    Xn = jnp.dot(t, mmat_ref[...], preferred_element_type=_F32)
    a_sn = jnp.dot(Xn, vst_ref[...], preferred_element_type=_F32)
    a_dn = jnp.dot(Xn, vdt_ref[...], preferred_element_type=_F32)
    nb = Xn.shape[0]
    xn_ref[...] = Xn
    g_ref[:, 0:32] = Xn
    g_ref[:, 32:36] = a_sn
    g_ref[:, 36:40] = jnp.zeros((nb, 4), _F32)
    ad_ref[:, 0:4] = a_dn
    ad_ref[:, 4:16] = jnp.zeros((nb, 12), _F32)
    asadn_ref[:, 0:4] = a_sn
    asadn_ref[:, 4:8] = a_dn


def _decode_body(x_ref, batch_ref, wd_ref, bd_ref, o_ref):
    s = jnp.dot(x_ref[...], wd_ref[...], preferred_element_type=_F32)
    gid = lax.broadcasted_iota(jnp.int32, (_N, _NGRAPHS), 1)
    oh = (batch_ref[...] == gid).astype(_F32)
    pooled = lax.dot_general(s, oh, (((0,), (0,)), ((), ())),
                             preferred_element_type=_F32)
    counts = jnp.sum(oh, axis=0, keepdims=True)
    o_ref[...] = pooled + bd_ref[0, 0] * counts


@functools.partial(
    pl.kernel,
    out_type=jax.ShapeDtypeStruct((2, _N, _ZW), _F32),
    mesh=plsc.VectorSubcoreMesh(core_axis_name="c", subcore_axis_name="s"),
    scratch_types=[
        pltpu.VMEM((2, _K), jnp.int32),
        pltpu.VMEM((2, _K), jnp.int32),
        pltpu.VMEM((_K, _GW), _F32),
        pltpu.VMEM((_K, _GW), _F32),
        pltpu.VMEM((_K, _ADW), _F32),
        pltpu.VMEM((_K, _ADW), _F32),
        pltpu.VMEM((_K, _ZW), _F32),
        pltpu.VMEM((_K, _ZW), _F32),
        pltpu.VMEM((1, _K), jnp.int32),
        pltpu.VMEM((1, _K), jnp.int32),
        pltpu.VMEM_SHARED((_N, _ZW), _F32),
        pltpu.SemaphoreType.DMA,
        pltpu.SemaphoreType.DMA,
        pltpu.SemaphoreType.DMA,
        pltpu.SemaphoreType.DMA,
        pltpu.SemaphoreType.DMA,
        pltpu.SemaphoreType.DMA,
        pltpu.SemaphoreType.DMA,
        pltpu.SemaphoreType.DMA,
    ],
    compiler_params=pltpu.CompilerParams(use_tc_tiling_on_sc=False,
                                         needs_layout_passes=False),
)
def _sc_edge_kernel(g_hbm, ad_hbm, ei_hbm, out_hbm,
                    ei0, ei1, g0, g1, ad0, ad1, wz0, wz1, si0, si1, acc,
                    sei0, sei1, sg0, sg1, sad0, sad1, ssc0, ssc1):
    c = lax.axis_index("c")
    s = lax.axis_index("s")
    tile = c * 16 + s
    eib = (ei0, ei1)
    gb = (g0, g1)
    adb = (ad0, ad1)
    wzb = (wz0, wz1)
    sib = (si0, si1)
    seib = (sei0, sei1)
    sgb = (sg0, sg1)
    sadb = (sad0, sad1)
    sscb = (ssc0, ssc1)

    for wz in (wz0, wz1):
        @pl.loop(0, _K)
        def _zero_wz(e, wz=wz):
            for j in (0, 16, 32, 48, 64, 80, 96, 112, 120):
                wz[e, pl.ds(j, 16)] = jnp.zeros((16,), _F32)
    wz = wz0

    base_row = s * _ROWS_PER_TILE
    nfull = _ROWS_PER_TILE // _K
    rem = _ROWS_PER_TILE % _K

    @pl.loop(0, nfull)
    def _zero_acc(i):
        pltpu.sync_copy(wz, acc.at[pl.ds(base_row + i * _K, _K)])

    pltpu.sync_copy(wz.at[pl.ds(0, rem)],
                    acc.at[pl.ds(base_row + nfull * _K, rem)])
    plsc.subcore_barrier()

    lanes = lax.iota(jnp.int32, 16)
    e4 = lanes // 4
    h4 = lanes - 4 * e4

    cnt = jnp.where(tile < 4, 158, 156)
    start = 156 * tile + 2 * jnp.minimum(tile, 4)

    def _load_ei(ch, b):
        pltpu.async_copy(ei_hbm.at[:, pl.ds(ch * _K, _K)], eib[b], seib[b])

    def _wait_ei(b):
        pltpu.make_async_copy(ei_hbm.at[:, pl.ds(0, _K)], eib[b],
                              seib[b]).wait()

    def _start_gathers(b):
        pltpu.async_copy(g_hbm.at[eib[b].at[0]], gb[b], sgb[b])
        pltpu.async_copy(ad_hbm.at[eib[b].at[1]], adb[b], sadb[b])

    def _wait_gathers(b):
        pltpu.make_async_copy(g_hbm.at[eib[b].at[0]], gb[b], sgb[b]).wait()
        pltpu.make_async_copy(ad_hbm.at[eib[b].at[1]], adb[b], sadb[b]).wait()

    _load_ei(start, 0)
    _wait_ei(0)
    _start_gathers(0)
    _load_ei(start + 1, 1)

    @pl.loop(0, cnt, step=2)
    def _chunk(i):
        for b in (0, 1):
            cur = i + b
            _wait_gathers(b)

            @pl.when(cur + 1 < cnt)
            def _prefetch():
                _wait_ei(1 - b)
                _start_gathers(1 - b)

            @pl.when(cur >= 2)
            def _drain_sc():
                pltpu.make_async_copy(wzb[b], acc.at[sib[b].at[0]],
                                      sscb[b]).wait()

            @pl.loop(0, _K // 4, unroll=2)
            def _wgrp(g):
                row_idx = g * 4 + e4
                a_s = plsc.load_gather(gb[b], [row_idx, 32 + h4])
                a_d = plsc.load_gather(adb[b], [row_idx, h4])
                e = a_s + a_d
                e = jnp.where(e > 0.0, e, 0.2 * e)
                wv = jnp.exp(e)
                plsc.store_scatter(wzb[b], [row_idx, 128 + h4], wv)
                g4 = g * 4
                for ii in range(4):
                    x0 = gb[b][g4 + ii, pl.ds(0, 16)]
                    x1 = gb[b][g4 + ii, pl.ds(16, 16)]
                    for h in range(4):
                        wh = wv[4 * ii + h]
                        wzb[b][g4 + ii, pl.ds(h * 32, 16)] = x0 * wh
                        wzb[b][g4 + ii, pl.ds(h * 32 + 16, 16)] = x1 * wh

            for j in range(0, _K, 16):
                sib[b][0, pl.ds(j, 16)] = eib[b][1, pl.ds(j, 16)]

            pltpu.async_copy(wzb[b], acc.at[sib[b].at[0]], sscb[b], add=True)

            @pl.when(cur + 2 < cnt)
            def _next_ei():
                _load_ei(start + cur + 2, b)

    for b in (0, 1):
        pltpu.make_async_copy(wzb[b], acc.at[sib[b].at[0]], sscb[b]).wait()
    plsc.subcore_barrier()
    pltpu.sync_copy(acc.at[pl.ds(base_row, _ROWS_PER_TILE)],
                    out_hbm.at[c, pl.ds(base_row, _ROWS_PER_TILE)])


def kernel(x, pos, edge_index, batch, W_enc, b_enc, W_res, b_res,
           W_gat, att_src, att_dst, b_gat, W_dec, b_dec):
    Wg3 = W_gat.reshape(_HEADS, _NHID, _NHID)
    vst = jnp.einsum("hj,hjk->kh", att_src, Wg3)
    vdt = jnp.einsum("hj,hjk->kh", att_dst, Wg3)
    wbd = jax.scipy.linalg.block_diag(*[Wg3[h].T for h in range(_HEADS)])
    sel = jnp.repeat(jnp.eye(_HEADS, dtype=_F32), _NHID, axis=1)
    mmat = jnp.repeat(jnp.eye(_NHID, dtype=_F32), _HEADS, axis=0) * 0.25
    bgat = b_gat.reshape(1, -1)
    bres = b_res.reshape(1, -1)
    wres_t = W_res.T

    inp = jnp.concatenate([x, pos], axis=1)
    node_shapes = (
        jax.ShapeDtypeStruct((_N, _NHID), _F32),
        jax.ShapeDtypeStruct((_N, _GW), _F32),
        jax.ShapeDtypeStruct((_N, _ADW), _F32),
        jax.ShapeDtypeStruct((_N, 8), _F32),
    )
    _NB = 2000
    _GRID = _N // _NB

    def _rows(w):
        return pl.BlockSpec((_NB, w), lambda i: (i, 0))

    def _full(a, b):
        return pl.BlockSpec((a, b), lambda i: (0, 0))

    node_out_specs = (_rows(_NHID), _rows(_GW), _rows(_ADW), _rows(8))
    X, G, AD, ASAD = pl.pallas_call(
        _encode_body, out_shape=node_shapes,
        grid=(_GRID,),
        in_specs=[_rows(128), _full(128, _NHID), _full(1, _NHID),
                  _full(_NHID, _HEADS), _full(_NHID, _HEADS)],
        out_specs=node_out_specs,
    )(inp, W_enc.T, b_enc.reshape(1, -1), vst, vdt)

    for _ in range(2):
        zpart = _sc_edge_kernel(G, AD, edge_index)
        X, G, AD, ASAD = pl.pallas_call(
            _combine_body, out_shape=node_shapes,
            grid=(_GRID,),
            in_specs=[pl.BlockSpec((1, _NB, _ZW), lambda i: (0, i, 0)),
                      pl.BlockSpec((1, _NB, _ZW), lambda i: (1, i, 0)),
                      _rows(_NHID), _rows(8),
                      _full(128, 128), _full(1, 128), _full(_NHID, 128),
                      _full(1, 128), _full(_NHID, _HEADS),
                      _full(_NHID, _HEADS), _full(_HEADS, 128),
                      _full(128, _NHID)],
            out_specs=node_out_specs,
        )(zpart, zpart, X, ASAD, wbd, bgat, wres_t, bres, vst, vdt, sel, mmat)

    out = pl.pallas_call(
        _decode_body,
        out_shape=jax.ShapeDtypeStruct((1, _NGRAPHS), _F32),
    )(X, batch.reshape(-1, 1), W_dec.T, b_dec.reshape(1, 1))
    return out.reshape(_NGRAPHS)
